# Initial kernel scaffold; baseline (speedup 1.0000x reference)
import jax
import jax.numpy as jnp
from jax.experimental import pallas as pl


def kernel(center_vecs, id2center, doc_ids, neg_ids):
    raise NotImplementedError("write your pallas kernel here")


# speedup vs baseline: 1.3364x; 1.3364x over previous
import functools

import jax
import jax.numpy as jnp
from jax import lax
from jax.experimental import pallas as pl
from jax.experimental.pallas import tpu as pltpu
from jax.experimental.pallas import tpu_sc as plsc

N_CENTERS = 65536
N_DOCS = 1000000
DIM = 128
BATCH = 16384

NC = 2
NS = 16
NW = NC * NS
CHUNK = 128
B_PER_W = BATCH // NW
CPW = B_PER_W // CHUNK


def _body(center_hbm, i2c_hbm, doc_hbm, neg_hbm, out_d_hbm, out_n_hbm,
          ids_v, cids_v, rows_v, sem):
    wid = lax.axis_index("s") * NC + lax.axis_index("c")
    base = wid * B_PER_W
    for ids_hbm, out_hbm in ((doc_hbm, out_d_hbm), (neg_hbm, out_n_hbm)):
        for j in range(CPW):
            pltpu.sync_copy(ids_hbm.at[wid, j], ids_v)
            pltpu.async_copy(i2c_hbm.at[ids_v], cids_v, sem).wait()
            pltpu.async_copy(center_hbm.at[cids_v], rows_v, sem).wait()
            pltpu.sync_copy(rows_v, out_hbm.at[pl.ds(base + j * CHUNK, CHUNK)])


@jax.jit
def _gather(center_vecs, id2center, doc_r, neg_r):
    mesh = plsc.VectorSubcoreMesh(core_axis_name="c", subcore_axis_name="s")
    out_sds = jax.ShapeDtypeStruct((BATCH, DIM), jnp.float32)
    k = pl.kernel(
        _body,
        mesh=mesh,
        out_type=(out_sds, out_sds),
        scratch_types=[
            pltpu.VMEM((CHUNK,), jnp.int32),
            pltpu.VMEM((CHUNK,), jnp.int32),
            pltpu.VMEM((CHUNK, DIM), jnp.float32),
            pltpu.SemaphoreType.DMA,
        ],
    )
    return k(center_vecs, id2center, doc_r, neg_r)


def kernel(center_vecs, id2center, doc_ids, neg_ids):
    doc_r = doc_ids.astype(jnp.int32).reshape(NW, CPW, CHUNK)
    neg_r = neg_ids.astype(jnp.int32).reshape(NW, CPW, CHUNK)
    return _gather(center_vecs, id2center.astype(jnp.int32), doc_r, neg_r)


# speedup vs baseline: 1.7899x; 1.3394x over previous
import jax
import jax.numpy as jnp
from jax import lax
from jax.experimental import pallas as pl
from jax.experimental.pallas import tpu as pltpu
from jax.experimental.pallas import tpu_sc as plsc

N_CENTERS = 65536
N_DOCS = 1000000
DIM = 128
BATCH = 16384

NC = 2
NS = 16
NW = NC * NS
CHUNK = 128
B_PER_W = BATCH // NW
CPW = B_PER_W // CHUNK
NCHUNK = 2 * CPW
NBUF = 4


def _body(center_hbm, i2c_hbm, doc_hbm, neg_hbm, out_d_hbm, out_n_hbm,
          ids_v, cids_v, rows_v, csem, gsem, ssem):
    wid = lax.axis_index("s") * NC + lax.axis_index("c")
    base = wid * B_PER_W

    pltpu.sync_copy(doc_hbm.at[wid], ids_v.at[pl.ds(0, CPW)])
    pltpu.sync_copy(neg_hbm.at[wid], ids_v.at[pl.ds(CPW, CPW)])
    cid_waits = []
    for j in range(NCHUNK):
        cid_waits.append(
            pltpu.async_copy(i2c_hbm.at[ids_v.at[j]], cids_v.at[j], csem))
    for w in cid_waits:
        w.wait()

    def out_slice(j):
        out_hbm = out_d_hbm if j < CPW else out_n_hbm
        return out_hbm.at[pl.ds(base + (j % CPW) * CHUNK, CHUNK)]

    gw = [None] * NBUF
    sw = [None] * NBUF
    for j in range(min(NBUF, NCHUNK)):
        gw[j] = pltpu.async_copy(center_hbm.at[cids_v.at[j]],
                                 rows_v.at[j], gsem.at[j])
    for j in range(NCHUNK):
        s = j % NBUF
        gw[s].wait()
        sw[s] = pltpu.async_copy(rows_v.at[s], out_slice(j), ssem.at[s])
        nxt = j + NBUF
        if nxt < NCHUNK:
            sw[s].wait()
            gw[s] = pltpu.async_copy(center_hbm.at[cids_v.at[nxt]],
                                     rows_v.at[s], gsem.at[s])
    for j in range(max(0, NCHUNK - NBUF), NCHUNK):
        sw[j % NBUF].wait()


@jax.jit
def _gather(center_vecs, id2center, doc_r, neg_r):
    mesh = plsc.VectorSubcoreMesh(core_axis_name="c", subcore_axis_name="s")
    out_sds = jax.ShapeDtypeStruct((BATCH, DIM), jnp.float32)
    k = pl.kernel(
        _body,
        mesh=mesh,
        out_type=(out_sds, out_sds),
        scratch_types=[
            pltpu.VMEM((NCHUNK, CHUNK), jnp.int32),
            pltpu.VMEM((NCHUNK, CHUNK), jnp.int32),
            pltpu.VMEM((NBUF, CHUNK, DIM), jnp.float32),
            pltpu.SemaphoreType.DMA,
            pltpu.SemaphoreType.DMA((NBUF,)),
            pltpu.SemaphoreType.DMA((NBUF,)),
        ],
    )
    return k(center_vecs, id2center, doc_r, neg_r)


def kernel(center_vecs, id2center, doc_ids, neg_ids):
    doc_r = doc_ids.astype(jnp.int32).reshape(NW, CPW, CHUNK)
    neg_r = neg_ids.astype(jnp.int32).reshape(NW, CPW, CHUNK)
    return _gather(center_vecs, id2center.astype(jnp.int32), doc_r, neg_r)


# speedup vs baseline: 1.8415x; 1.0288x over previous
import jax
import jax.numpy as jnp
from jax import lax
from jax.experimental import pallas as pl
from jax.experimental.pallas import tpu as pltpu
from jax.experimental.pallas import tpu_sc as plsc

N_CENTERS = 65536
N_DOCS = 1000000
DIM = 128
BATCH = 16384

NC = 2
NS = 16
NW = NC * NS
CHUNK = 128
B_PER_W = BATCH // NW
CPW = B_PER_W // CHUNK
NCHUNK = 2 * CPW
NBUF = 7


def _body(center_hbm, i2c_hbm, doc_hbm, neg_hbm, out_d_hbm, out_n_hbm,
          ids_v, cids_v, rows_v, csem, gsem, ssem):
    wid = lax.axis_index("s") * NC + lax.axis_index("c")
    base = wid * B_PER_W

    pltpu.sync_copy(doc_hbm.at[wid], ids_v.at[pl.ds(0, CPW)])
    pltpu.sync_copy(neg_hbm.at[wid], ids_v.at[pl.ds(CPW, CPW)])
    cw = [pltpu.async_copy(i2c_hbm.at[ids_v.at[j]], cids_v.at[j], csem.at[j])
          for j in range(NCHUNK)]

    def out_slice(j):
        out_hbm = out_d_hbm if j < CPW else out_n_hbm
        return out_hbm.at[pl.ds(base + (j % CPW) * CHUNK, CHUNK)]

    gw = [None] * NCHUNK
    sw = [None] * NCHUNK
    for j in range(min(NBUF, NCHUNK)):
        cw[j].wait()
        gw[j] = pltpu.async_copy(center_hbm.at[cids_v.at[j]],
                                 rows_v.at[j], gsem.at[j])
    for j in range(NCHUNK):
        s = j % NBUF
        if j >= NBUF:
            sw[j - NBUF].wait()
            cw[j].wait()
            gw[j] = pltpu.async_copy(center_hbm.at[cids_v.at[j]],
                                     rows_v.at[s], gsem.at[s])
        gw[j].wait()
        sw[j] = pltpu.async_copy(rows_v.at[s], out_slice(j), ssem.at[s])
    for j in range(max(0, NCHUNK - NBUF), NCHUNK):
        sw[j].wait()


@jax.jit
def _gather(center_vecs, id2center, doc_r, neg_r):
    mesh = plsc.VectorSubcoreMesh(core_axis_name="c", subcore_axis_name="s")
    out_sds = jax.ShapeDtypeStruct((BATCH, DIM), jnp.float32)
    k = pl.kernel(
        _body,
        mesh=mesh,
        out_type=(out_sds, out_sds),
        scratch_types=[
            pltpu.VMEM((NCHUNK, CHUNK), jnp.int32),
            pltpu.VMEM((NCHUNK, CHUNK), jnp.int32),
            pltpu.VMEM((NBUF, CHUNK, DIM), jnp.float32),
            pltpu.SemaphoreType.DMA((NCHUNK,)),
            pltpu.SemaphoreType.DMA((NBUF,)),
            pltpu.SemaphoreType.DMA((NBUF,)),
        ],
    )
    return k(center_vecs, id2center, doc_r, neg_r)


def kernel(center_vecs, id2center, doc_ids, neg_ids):
    doc_r = doc_ids.astype(jnp.int32).reshape(NW, CPW, CHUNK)
    neg_r = neg_ids.astype(jnp.int32).reshape(NW, CPW, CHUNK)
    return _gather(center_vecs, id2center.astype(jnp.int32), doc_r, neg_r)


# speedup vs baseline: 1.8621x; 1.0112x over previous
import jax
import jax.numpy as jnp
from jax import lax
from jax.experimental import pallas as pl
from jax.experimental.pallas import tpu as pltpu
from jax.experimental.pallas import tpu_sc as plsc

N_CENTERS = 65536
N_DOCS = 1000000
DIM = 128
BATCH = 16384

NC = 2
NS = 16
NW = NC * NS
CHUNK = 128
B_PER_W = BATCH // NW
CPW = B_PER_W // CHUNK
NCHUNK = 2 * CPW
NBUF = 7


def _body(center_hbm, i2c_hbm, doc_hbm, neg_hbm, out_d_hbm, out_n_hbm,
          ids_v, cids_v, rows_v, csem, gsem, ssem):
    wid = lax.axis_index("s") * NC + lax.axis_index("c")
    base = wid * B_PER_W

    pltpu.sync_copy(doc_hbm.at[wid], ids_v.at[pl.ds(0, CPW)])
    cw = [pltpu.async_copy(i2c_hbm.at[ids_v.at[j]], cids_v.at[j], csem.at[j])
          for j in range(CPW)]
    pltpu.sync_copy(neg_hbm.at[wid], ids_v.at[pl.ds(CPW, CPW)])
    cw += [pltpu.async_copy(i2c_hbm.at[ids_v.at[j]], cids_v.at[j], csem.at[j])
           for j in range(CPW, NCHUNK)]

    def out_slice(j):
        out_hbm = out_d_hbm if j < CPW else out_n_hbm
        return out_hbm.at[pl.ds(base + (j % CPW) * CHUNK, CHUNK)]

    gw = [None] * NCHUNK
    sw = [None] * NCHUNK
    for j in range(min(NBUF, NCHUNK)):
        cw[j].wait()
        gw[j] = pltpu.async_copy(center_hbm.at[cids_v.at[j]],
                                 rows_v.at[j], gsem.at[j])
    for j in range(NCHUNK):
        s = j % NBUF
        if j >= NBUF:
            sw[j - NBUF].wait()
            cw[j].wait()
            gw[j] = pltpu.async_copy(center_hbm.at[cids_v.at[j]],
                                     rows_v.at[s], gsem.at[s])
        gw[j].wait()
        sw[j] = pltpu.async_copy(rows_v.at[s], out_slice(j), ssem.at[s])
    for j in range(max(0, NCHUNK - NBUF), NCHUNK):
        sw[j].wait()


@jax.jit
def _gather(center_vecs, id2center, doc_r, neg_r):
    mesh = plsc.VectorSubcoreMesh(core_axis_name="c", subcore_axis_name="s")
    out_sds = jax.ShapeDtypeStruct((BATCH, DIM), jnp.float32)
    k = pl.kernel(
        _body,
        mesh=mesh,
        out_type=(out_sds, out_sds),
        scratch_types=[
            pltpu.VMEM((NCHUNK, CHUNK), jnp.int32),
            pltpu.VMEM((NCHUNK, CHUNK), jnp.int32),
            pltpu.VMEM((NBUF, CHUNK, DIM), jnp.float32),
            pltpu.SemaphoreType.DMA((NCHUNK,)),
            pltpu.SemaphoreType.DMA((NBUF,)),
            pltpu.SemaphoreType.DMA((NBUF,)),
        ],
    )
    return k(center_vecs, id2center, doc_r, neg_r)


def kernel(center_vecs, id2center, doc_ids, neg_ids):
    doc_r = doc_ids.astype(jnp.int32).reshape(NW, CPW, CHUNK)
    neg_r = neg_ids.astype(jnp.int32).reshape(NW, CPW, CHUNK)
    return _gather(center_vecs, id2center.astype(jnp.int32), doc_r, neg_r)
